# Initial kernel scaffold; baseline (speedup 1.0000x reference)
import jax
import jax.numpy as jnp
from jax.experimental import pallas as pl


def kernel(xyz, centroids, params):
    raise NotImplementedError("write your pallas kernel here")


# speedup vs baseline: 1.4426x; 1.4426x over previous
import functools

import jax
import jax.numpy as jnp
from jax.experimental import pallas as pl

_NPROP_K = 512


def _fps_kernel(x_ref, out_ref, *, npoint):
    B, _, N = x_ref.shape
    x = x_ref[:, 0, :]
    y = x_ref[:, 1, :]
    z = x_ref[:, 2, :]
    iota = jax.lax.broadcasted_iota(jnp.int32, (B, N), 1)
    iota_np = jax.lax.broadcasted_iota(jnp.int32, (B, npoint), 1)

    out_ref[:, :] = jnp.zeros((B, npoint), jnp.int32)

    def body(i, carry):
        dist, far = carry
        farb = jnp.broadcast_to(far, (B, npoint))
        out_ref[:, :] = out_ref[:, :] + jnp.where(iota_np == i, farb, 0)
        mask = iota == far
        cx = jnp.sum(jnp.where(mask, x, 0.0), axis=1, keepdims=True)
        cy = jnp.sum(jnp.where(mask, y, 0.0), axis=1, keepdims=True)
        cz = jnp.sum(jnp.where(mask, z, 0.0), axis=1, keepdims=True)
        d = (x - cx) ** 2 + (y - cy) ** 2 + (z - cz) ** 2
        dist = jnp.minimum(dist, d)
        m = jnp.max(dist, axis=1, keepdims=True)
        far = jnp.min(jnp.where(dist == m, iota, N), axis=1, keepdims=True)
        return dist, far

    dist0 = jnp.full((B, N), 1e10, jnp.float32)
    far0 = jnp.zeros((B, 1), jnp.int32)
    jax.lax.fori_loop(0, npoint, body, (dist0, far0))


def _fps(xyz3, npoint):
    B, _, N = xyz3.shape
    return pl.pallas_call(
        functools.partial(_fps_kernel, npoint=npoint),
        out_shape=jax.ShapeDtypeStruct((B, npoint), jnp.int32),
    )(xyz3)


def _square_distance(src, dst):
    return (
        jnp.sum(src ** 2, -1)[:, :, None]
        + jnp.sum(dst ** 2, -1)[:, None, :]
        - 2.0 * jnp.einsum('bnc,bmc->bnm', src, dst)
    )


def _index_points(points, idx):
    return jax.vmap(lambda p, i: p[i])(points, idx)


def _query_ball_point(radius, nsample, xyz, new_xyz):
    B, N, _ = xyz.shape
    S = new_xyz.shape[1]
    sqrdists = _square_distance(new_xyz, xyz)
    gi = jnp.broadcast_to(jnp.arange(N, dtype=jnp.int32), (B, S, N))
    gi = jnp.where(sqrdists > radius ** 2, N, gi)
    gi = jnp.sort(gi, axis=-1)[:, :, :nsample]
    first = gi[:, :, :1]
    gi = jnp.where(gi == N, first, gi)
    return gi


def _bn(x, gamma, beta, axes):
    mean = jnp.mean(x, axis=axes, keepdims=True)
    var = jnp.var(x, axis=axes, keepdims=True)
    shape = [1] * x.ndim
    shape[1] = -1
    return (x - mean) / jnp.sqrt(var + 1e-5) * gamma.reshape(shape) + beta.reshape(shape)


def _sa_msg(branches, npoint, radius_list, nsample_list, xyz, points):
    xyz_t = jnp.transpose(xyz, (0, 2, 1))
    pts_t = jnp.transpose(points, (0, 2, 1))
    fps_idx = _fps(jax.lax.stop_gradient(xyz), npoint)
    new_xyz = _index_points(xyz_t, fps_idx)
    outs = []
    for layers, radius, K in zip(branches, radius_list, nsample_list):
        gi = _query_ball_point(radius, K, xyz_t, new_xyz)
        gxyz = _index_points(xyz_t, gi) - new_xyz[:, :, None, :]
        g = jnp.concatenate([_index_points(pts_t, gi), gxyz], axis=-1)
        g = jnp.transpose(g, (0, 3, 2, 1))
        for (W, b, gamma, beta) in layers:
            g = jnp.einsum('oc,bcks->boks', W, g) + b[None, :, None, None]
            g = jax.nn.relu(_bn(g, gamma, beta, (0, 2, 3)))
        outs.append(jnp.max(g, axis=2))
    return jnp.transpose(new_xyz, (0, 2, 1)), jnp.concatenate(outs, axis=1)


def _fp(layers, xyz1, xyz2, points1, points2):
    x1 = jnp.transpose(xyz1, (0, 2, 1))
    x2 = jnp.transpose(xyz2, (0, 2, 1))
    p2 = jnp.transpose(points2, (0, 2, 1))
    d = _square_distance(x1, x2)
    idx = jnp.argsort(d, axis=-1)[:, :, :3]
    d3 = jnp.take_along_axis(d, idx, axis=-1)
    rec = 1.0 / (d3 + 1e-8)
    w = rec / jnp.sum(rec, axis=-1, keepdims=True)
    interp = jnp.sum(_index_points(p2, idx) * w[..., None], axis=2)
    if points1 is not None:
        newp = jnp.concatenate([jnp.transpose(points1, (0, 2, 1)), interp], axis=-1)
    else:
        newp = interp
    g = jnp.transpose(newp, (0, 2, 1))
    for (W, b, gamma, beta) in layers:
        g = jnp.einsum('oc,bcn->bon', W, g) + b[None, :, None]
        g = jax.nn.relu(_bn(g, gamma, beta, (0, 2)))
    return g


def _get_proposal(xyz, points, centroids, nsample=_NPROP_K):
    xt = jnp.transpose(xyz, (0, 2, 1))
    ct = jnp.transpose(centroids, (0, 2, 1))
    d = _square_distance(ct, xt)
    neg, idx = jax.lax.top_k(-d, nsample)
    gxyz = _index_points(xt, idx) - ct[:, :, None, :]
    dist = jnp.sqrt(jnp.maximum(-neg, 0.0))[..., None]
    gpts = _index_points(jnp.transpose(points, (0, 2, 1)), idx)
    props = jnp.concatenate([gxyz, dist, gpts], axis=-1)
    return jnp.transpose(props, (0, 1, 3, 2)), idx


def kernel(xyz, centroids, params):
    l0_points = xyz
    l0_xyz = xyz[:, :3, :]
    l1_xyz, l1_points = _sa_msg(params['sa1'], 1024, [2.5, 5.0], [16, 32], l0_xyz, l0_points)
    l2_xyz, l2_points = _sa_msg(params['sa2'], 512, [5.0, 10.0], [16, 32], l1_xyz, l1_points)
    l3_xyz, l3_points = _sa_msg(params['sa3'], 256, [10.0, 20.0], [16, 32], l2_xyz, l2_points)
    l2_points = _fp(params['fp3'], l2_xyz, l3_xyz, l2_points, l3_points)
    l1_points = _fp(params['fp2'], l1_xyz, l2_xyz, l1_points, l2_points)
    l0_points = _fp(params['fp1'], l0_xyz, l1_xyz, l0_points, l1_points)
    cen = jnp.transpose(centroids.reshape(-1, 14, 3), (0, 2, 1))
    proposals, proposal_index = _get_proposal(l0_xyz, l0_points, cen)
    B = l0_points.shape[0]
    K = proposals.shape[-1]
    p = proposals.reshape(-1, proposals.shape[-2], K)
    W1, b1, g1, be1 = params['conv1']
    x = jnp.einsum('oc,bck->bok', W1, p) + b1[None, :, None]
    x = jax.nn.relu(_bn(x, g1, be1, (0, 2)))
    W2, b2 = params['conv2']
    x = jnp.einsum('oc,bck->bok', W2, x) + b2[None, :, None]
    x = jax.nn.log_softmax(x, axis=1)
    x = jnp.transpose(x, (0, 2, 1)).reshape(B, -1, K, 16)
    return x, proposal_index


# speedup vs baseline: 2.3974x; 1.6619x over previous
import functools

import jax
import jax.numpy as jnp
from jax.experimental import pallas as pl

_NPROP_K = 512


def _fps_kernel(x_ref, out_ref, *, npoint):
    B, _, N = x_ref.shape
    x = x_ref[:, 0, :]
    y = x_ref[:, 1, :]
    z = x_ref[:, 2, :]
    iota = jax.lax.broadcasted_iota(jnp.int32, (B, N), 1)
    iota_np = jax.lax.broadcasted_iota(jnp.int32, (B, npoint), 1)

    out_ref[:, :] = jnp.zeros((B, npoint), jnp.int32)

    def body(i, carry):
        dist, far = carry
        farb = jnp.broadcast_to(far, (B, npoint))
        out_ref[:, :] = out_ref[:, :] + jnp.where(iota_np == i, farb, 0)
        mask = iota == far
        cx = jnp.sum(jnp.where(mask, x, 0.0), axis=1, keepdims=True)
        cy = jnp.sum(jnp.where(mask, y, 0.0), axis=1, keepdims=True)
        cz = jnp.sum(jnp.where(mask, z, 0.0), axis=1, keepdims=True)
        dx = x - cx
        dy = y - cy
        dz = z - cz
        d = dx * dx + (dy * dy + dz * dz)
        dist = jnp.minimum(dist, d)
        m = jnp.max(dist, axis=1, keepdims=True)
        far = jnp.min(jnp.where(dist == m, iota, N), axis=1, keepdims=True)
        return dist, far

    dist0 = jnp.full((B, N), 1e10, jnp.float32)
    far0 = jnp.zeros((B, 1), jnp.int32)
    jax.lax.fori_loop(0, npoint, body, (dist0, far0))


def _fps(xyz3, npoint):
    B, _, N = xyz3.shape
    return pl.pallas_call(
        functools.partial(_fps_kernel, npoint=npoint),
        out_shape=jax.ShapeDtypeStruct((B, npoint), jnp.int32),
    )(xyz3)


def _ballq_kernel(d_ref, out_ref, *, r2, K):
    _, TS, N = d_ref.shape
    dd = d_ref[0]
    iota = jax.lax.broadcasted_iota(jnp.int32, (TS, N), 1)
    cand = jnp.where(dd > r2, N, iota)
    cols = []
    prev = jnp.full((TS, 1), -1, jnp.int32)
    for _ in range(K):
        cur = jnp.min(jnp.where(cand > prev, cand, N), axis=1, keepdims=True)
        cols.append(cur)
        prev = cur
    g = jnp.concatenate(cols, axis=1)
    first = jnp.broadcast_to(cols[0], (TS, K))
    out_ref[0] = jnp.where(g == N, first, g)


def _ball_query_select(d, radius, K):
    B, S, N = d.shape
    TS = min(S, 256)
    return pl.pallas_call(
        functools.partial(_ballq_kernel, r2=radius ** 2, K=K),
        grid=(B, S // TS),
        in_specs=[pl.BlockSpec((1, TS, N), lambda b, s: (b, s, 0))],
        out_specs=pl.BlockSpec((1, TS, K), lambda b, s: (b, s, 0)),
        out_shape=jax.ShapeDtypeStruct((B, S, K), jnp.int32),
    )(d)


def _knn3_kernel(d_ref, idx_ref, val_ref, *, K):
    _, TS, N = d_ref.shape
    dd = d_ref[0]
    iota = jax.lax.broadcasted_iota(jnp.int32, (TS, N), 1)
    active = iota == iota
    icols, vcols = [], []
    for _ in range(K):
        dmask = jnp.where(active, dd, jnp.inf)
        v = jnp.min(dmask, axis=1, keepdims=True)
        i = jnp.min(jnp.where(dmask == v, iota, N), axis=1, keepdims=True)
        icols.append(i)
        vcols.append(v)
        active = jnp.logical_and(active, iota != i)
    idx_ref[0] = jnp.concatenate(icols, axis=1)
    val_ref[0] = jnp.concatenate(vcols, axis=1)


def _knn3_select(d, K=3):
    B, N1, N2 = d.shape
    TS = min(N1, 512)
    return pl.pallas_call(
        functools.partial(_knn3_kernel, K=K),
        grid=(B, N1 // TS),
        in_specs=[pl.BlockSpec((1, TS, N2), lambda b, s: (b, s, 0))],
        out_specs=[
            pl.BlockSpec((1, TS, K), lambda b, s: (b, s, 0)),
            pl.BlockSpec((1, TS, K), lambda b, s: (b, s, 0)),
        ],
        out_shape=[
            jax.ShapeDtypeStruct((B, N1, K), jnp.int32),
            jax.ShapeDtypeStruct((B, N1, K), jnp.float32),
        ],
    )(d)


def _square_distance(src, dst):
    return (
        jnp.sum(src ** 2, -1)[:, :, None]
        + jnp.sum(dst ** 2, -1)[:, None, :]
        - 2.0 * jnp.einsum('bnc,bmc->bnm', src, dst)
    )


def _index_points(points, idx):
    return jax.vmap(lambda p, i: p[i])(points, idx)


def _query_ball_point(radius, nsample, xyz, new_xyz):
    sqrdists = _square_distance(new_xyz, xyz)
    return _ball_query_select(sqrdists, radius, nsample)


def _bn(x, gamma, beta, axes):
    mean = jnp.mean(x, axis=axes, keepdims=True)
    var = jnp.var(x, axis=axes, keepdims=True)
    shape = [1] * x.ndim
    shape[1] = -1
    return (x - mean) / jnp.sqrt(var + 1e-5) * gamma.reshape(shape) + beta.reshape(shape)


def _sa_msg(branches, npoint, radius_list, nsample_list, xyz, points):
    xyz_t = jnp.transpose(xyz, (0, 2, 1))
    pts_t = jnp.transpose(points, (0, 2, 1))
    fps_idx = _fps(jax.lax.stop_gradient(xyz), npoint)
    new_xyz = _index_points(xyz_t, fps_idx)
    outs = []
    for layers, radius, K in zip(branches, radius_list, nsample_list):
        gi = _query_ball_point(radius, K, xyz_t, new_xyz)
        gxyz = _index_points(xyz_t, gi) - new_xyz[:, :, None, :]
        g = jnp.concatenate([_index_points(pts_t, gi), gxyz], axis=-1)
        g = jnp.transpose(g, (0, 3, 2, 1))
        for (W, b, gamma, beta) in layers:
            g = jnp.einsum('oc,bcks->boks', W, g) + b[None, :, None, None]
            g = jax.nn.relu(_bn(g, gamma, beta, (0, 2, 3)))
        outs.append(jnp.max(g, axis=2))
    return jnp.transpose(new_xyz, (0, 2, 1)), jnp.concatenate(outs, axis=1)


def _fp(layers, xyz1, xyz2, points1, points2):
    x1 = jnp.transpose(xyz1, (0, 2, 1))
    x2 = jnp.transpose(xyz2, (0, 2, 1))
    p2 = jnp.transpose(points2, (0, 2, 1))
    d = _square_distance(x1, x2)
    idx, d3 = _knn3_select(d)
    rec = 1.0 / (d3 + 1e-8)
    w = rec / jnp.sum(rec, axis=-1, keepdims=True)
    interp = jnp.sum(_index_points(p2, idx) * w[..., None], axis=2)
    if points1 is not None:
        newp = jnp.concatenate([jnp.transpose(points1, (0, 2, 1)), interp], axis=-1)
    else:
        newp = interp
    g = jnp.transpose(newp, (0, 2, 1))
    for (W, b, gamma, beta) in layers:
        g = jnp.einsum('oc,bcn->bon', W, g) + b[None, :, None]
        g = jax.nn.relu(_bn(g, gamma, beta, (0, 2)))
    return g


def _get_proposal(xyz, points, centroids, nsample=_NPROP_K):
    xt = jnp.transpose(xyz, (0, 2, 1))
    ct = jnp.transpose(centroids, (0, 2, 1))
    d = _square_distance(ct, xt)
    neg, idx = jax.lax.top_k(-d, nsample)
    gxyz = _index_points(xt, idx) - ct[:, :, None, :]
    dist = jnp.sqrt(jnp.maximum(-neg, 0.0))[..., None]
    gpts = _index_points(jnp.transpose(points, (0, 2, 1)), idx)
    props = jnp.concatenate([gxyz, dist, gpts], axis=-1)
    return jnp.transpose(props, (0, 1, 3, 2)), idx


def kernel(xyz, centroids, params):
    l0_points = xyz
    l0_xyz = xyz[:, :3, :]
    l1_xyz, l1_points = _sa_msg(params['sa1'], 1024, [2.5, 5.0], [16, 32], l0_xyz, l0_points)
    l2_xyz, l2_points = _sa_msg(params['sa2'], 512, [5.0, 10.0], [16, 32], l1_xyz, l1_points)
    l3_xyz, l3_points = _sa_msg(params['sa3'], 256, [10.0, 20.0], [16, 32], l2_xyz, l2_points)
    l2_points = _fp(params['fp3'], l2_xyz, l3_xyz, l2_points, l3_points)
    l1_points = _fp(params['fp2'], l1_xyz, l2_xyz, l1_points, l2_points)
    l0_points = _fp(params['fp1'], l0_xyz, l1_xyz, l0_points, l1_points)
    cen = jnp.transpose(centroids.reshape(-1, 14, 3), (0, 2, 1))
    proposals, proposal_index = _get_proposal(l0_xyz, l0_points, cen)
    B = l0_points.shape[0]
    K = proposals.shape[-1]
    p = proposals.reshape(-1, proposals.shape[-2], K)
    W1, b1, g1, be1 = params['conv1']
    x = jnp.einsum('oc,bck->bok', W1, p) + b1[None, :, None]
    x = jax.nn.relu(_bn(x, g1, be1, (0, 2)))
    W2, b2 = params['conv2']
    x = jnp.einsum('oc,bck->bok', W2, x) + b2[None, :, None]
    x = jax.nn.log_softmax(x, axis=1)
    x = jnp.transpose(x, (0, 2, 1)).reshape(B, -1, K, 16)
    return x, proposal_index
